# baseline (speedup 1.0000x reference)
import jax
import jax.numpy as jnp
from jax.experimental import pallas as pl
from jax.experimental.pallas import tpu as pltpu

_B, _N, _C = 2, 4096, 128
_TM = 512
_NT = _N // _TM


def _fused_kernel(x_ref, a_ref, aW1_ref, ab1_ref, aW2_ref, ab2_ref,
                  aW3_ref, ab3_ref, gW1_ref, gb1_ref, gW2_ref, gb2_ref,
                  gW3_ref, gb3_ref, out_ref, wf_s, gc_s, ds_s, sc_s):
    b = pl.program_id(0)
    i = pl.program_id(1)

    def scores(xx):
        h = jnp.maximum(
            jnp.dot(xx, aW1_ref[...], preferred_element_type=jnp.float32)
            + ab1_ref[...], 0.0)
        h = jnp.maximum(
            jnp.dot(h, aW2_ref[...], preferred_element_type=jnp.float32)
            + ab2_ref[...], 0.0)
        return jnp.dot(h, aW3_ref[...], preferred_element_type=jnp.float32) \
            + ab3_ref[...]

    def weight_store(bb, xx, s):
        s = s - jnp.max(s)
        e = jnp.exp(s)
        w = e / jnp.sum(e)
        wf = xx * w
        wf_s[bb] = wf
        gc_s[bb:bb + 1, :] = jnp.sum(wf, axis=0, keepdims=True)
        ds_s[bb:bb + 1, :] = jnp.zeros((1, _C), jnp.float32)

    @pl.when((b == 0) & (i == 0))
    def _attention_b0():
        xx = x_ref[0]
        weight_store(0, xx, scores(xx))

    @pl.when(b == 0)
    def _score_chunk_b1():
        xx = x_ref[pl.ds(1, 1), pl.ds(i * _TM, _TM), :][0]
        sc_s[pl.ds(i * _TM, _TM), :] = scores(xx)

    @pl.when((b == 1) & (i == 0))
    def _attention_b1():
        weight_store(1, x_ref[1], sc_s[...])

    wf_b = wf_s[pl.ds(b, 1)][0]
    m = jnp.dot(a_ref[0], wf_b, preferred_element_type=jnp.float32)
    wfr = wf_s[pl.ds(b, 1), pl.ds(i * _TM, _TM), :][0]
    ds_s[pl.ds(b, 1), :] += jnp.sum(m * wfr, axis=0, keepdims=True)

    @pl.when((b == _B - 1) & (i == _NT - 1))
    def _global_mlp():
        gc = gc_s[...]
        cs = jnp.sum(ds_s[...], axis=1, keepdims=True) * (1.0 / _N)
        hin = jnp.concatenate([gc, gc + 0.001 * cs], axis=0)
        h = jnp.maximum(
            jnp.dot(hin, gW1_ref[...], preferred_element_type=jnp.float32)
            + gb1_ref[...], 0.0)
        h = jnp.maximum(
            jnp.dot(h, gW2_ref[...], preferred_element_type=jnp.float32)
            + gb2_ref[...], 0.0)
        g = jnp.maximum(
            jnp.dot(h, gW3_ref[...], preferred_element_type=jnp.float32)
            + gb3_ref[...], 0.0)
        out_ref[...] = g[:_B] + 0.002 * g[_B:]


def kernel(x, adjacency, aW1, ab1, aW2, ab2, aW3, ab3,
           gW1, gb1, gW2, gb2, gW3, gb3):
    ab1r, ab2r, ab3r = ab1.reshape(1, -1), ab2.reshape(1, -1), ab3.reshape(1, -1)
    gb1r, gb2r, gb3r = gb1.reshape(1, -1), gb2.reshape(1, -1), gb3.reshape(1, -1)

    const = lambda shape: pl.BlockSpec(shape, lambda b, i: tuple(0 for _ in shape))
    out = pl.pallas_call(
        _fused_kernel,
        grid=(_B, _NT),
        in_specs=[
            const((_B, _N, _C)),
            pl.BlockSpec((1, _TM, _N), lambda b, i: (b, i, 0)),
            const((_C, 128)), const((1, 128)),
            const((128, 64)), const((1, 64)),
            const((64, 1)), const((1, 1)),
            const((_C, 1024)), const((1, 1024)),
            const((1024, 1024)), const((1, 1024)),
            const((1024, _C)), const((1, _C)),
        ],
        out_specs=pl.BlockSpec((_B, _C), lambda b, i: (0, 0)),
        out_shape=jax.ShapeDtypeStruct((_B, _C), jnp.float32),
        scratch_shapes=[
            pltpu.VMEM((_B, _N, _C), jnp.float32),
            pltpu.VMEM((_B, _C), jnp.float32),
            pltpu.VMEM((_B, _C), jnp.float32),
            pltpu.VMEM((_N, 1), jnp.float32),
        ],
    )(x, adjacency, aW1, ab1r, aW2, ab2r, aW3, ab3r,
      gW1, gb1r, gW2, gb2r, gW3, gb3r)
    return out


# speedup vs baseline: 1.0376x; 1.0376x over previous
import jax
import jax.numpy as jnp
from jax.experimental import pallas as pl
from jax.experimental.pallas import tpu as pltpu

_B, _N, _C = 2, 4096, 128
_TM = 512
_NT = _N // _TM


def _fused_kernel(x_ref, a_ref, aW1_ref, ab1_ref, aW2_ref, ab2_ref,
                  aW3_ref, ab3_ref, gW1_ref, gb1_ref, gW2_ref, gb2_ref,
                  gW3_ref, gb3_ref, out_ref, wf_s, gc_s, ds_s):
    b = pl.program_id(0)
    i = pl.program_id(1)

    def scores(xx):
        h = jnp.maximum(
            jnp.dot(xx, aW1_ref[...], preferred_element_type=jnp.float32)
            + ab1_ref[...], 0.0)
        h = jnp.maximum(
            jnp.dot(h, aW2_ref[...], preferred_element_type=jnp.float32)
            + ab2_ref[...], 0.0)
        return jnp.dot(h, aW3_ref[...], preferred_element_type=jnp.float32) \
            + ab3_ref[...]

    def weight_store(bb, xx, s):
        s = s - jnp.max(s)
        e = jnp.exp(s)
        w = e / jnp.sum(e)
        wf = xx * w
        wf_s[bb] = wf
        gc_s[bb:bb + 1, :] = jnp.sum(wf, axis=0, keepdims=True)
        ds_s[bb:bb + 1, :] = jnp.zeros((1, _C), jnp.float32)

    @pl.when((b == 0) & (i == 0))
    def _attention():
        for bb in range(_B):
            xx = x_ref[bb]
            weight_store(bb, xx, scores(xx))

    wf_b = wf_s[pl.ds(b, 1)][0]
    m = jnp.dot(a_ref[0], wf_b, preferred_element_type=jnp.float32)
    wfr = wf_s[pl.ds(b, 1), pl.ds(i * _TM, _TM), :][0]
    ds_s[pl.ds(b, 1), :] += jnp.sum(m * wfr, axis=0, keepdims=True)

    @pl.when((b == _B - 1) & (i == _NT - 1))
    def _global_mlp():
        gc = gc_s[...]
        cs = jnp.sum(ds_s[...], axis=1, keepdims=True) * (1.0 / _N)
        hin = jnp.concatenate([gc, gc + 0.001 * cs], axis=0)
        h = jnp.maximum(
            jnp.dot(hin, gW1_ref[...], preferred_element_type=jnp.float32)
            + gb1_ref[...], 0.0)
        h = jnp.maximum(
            jnp.dot(h, gW2_ref[...], preferred_element_type=jnp.float32)
            + gb2_ref[...], 0.0)
        g = jnp.maximum(
            jnp.dot(h, gW3_ref[...], preferred_element_type=jnp.float32)
            + gb3_ref[...], 0.0)
        out_ref[...] = g[:_B] + 0.002 * g[_B:]


def kernel(x, adjacency, aW1, ab1, aW2, ab2, aW3, ab3,
           gW1, gb1, gW2, gb2, gW3, gb3):
    ab1r, ab2r, ab3r = ab1.reshape(1, -1), ab2.reshape(1, -1), ab3.reshape(1, -1)
    gb1r, gb2r, gb3r = gb1.reshape(1, -1), gb2.reshape(1, -1), gb3.reshape(1, -1)

    const = lambda shape: pl.BlockSpec(shape, lambda b, i: tuple(0 for _ in shape))
    out = pl.pallas_call(
        _fused_kernel,
        grid=(_B, _NT),
        in_specs=[
            const((_B, _N, _C)),
            pl.BlockSpec((1, _TM, _N), lambda b, i: (b, i, 0)),
            const((_C, 128)), const((1, 128)),
            const((128, 64)), const((1, 64)),
            const((64, 1)), const((1, 1)),
            const((_C, 1024)), const((1, 1024)),
            const((1024, 1024)), const((1, 1024)),
            const((1024, _C)), const((1, _C)),
        ],
        out_specs=pl.BlockSpec((_B, _C), lambda b, i: (0, 0)),
        out_shape=jax.ShapeDtypeStruct((_B, _C), jnp.float32),
        scratch_shapes=[
            pltpu.VMEM((_B, _N, _C), jnp.float32),
            pltpu.VMEM((_B, _C), jnp.float32),
            pltpu.VMEM((_B, _C), jnp.float32),
        ],
    )(x, adjacency, aW1, ab1r, aW2, ab2r, aW3, ab3r,
      gW1, gb1r, gW2, gb2r, gW3, gb3r)
    return out


# speedup vs baseline: 1.0377x; 1.0001x over previous
import jax
import jax.numpy as jnp
from jax.experimental import pallas as pl
from jax.experimental.pallas import tpu as pltpu

_B, _N, _C = 2, 4096, 128
_TM = 512
_NT = _N // _TM


def _fused_kernel(x_ref, a_ref, aW1_ref, ab1_ref, aW2_ref, ab2_ref,
                  aW3_ref, ab3_ref, gW1_ref, gb1_ref, gW2_ref, gb2_ref,
                  gW3_ref, gb3_ref, out_ref, wf_s, gc_s, ds_s):
    b = pl.program_id(0)
    i = pl.program_id(1)

    def scores(xx):
        h = jnp.maximum(
            jnp.dot(xx, aW1_ref[...], preferred_element_type=jnp.float32)
            + ab1_ref[...][None, :], 0.0)
        h = jnp.maximum(
            jnp.dot(h, aW2_ref[...], preferred_element_type=jnp.float32)
            + ab2_ref[...][None, :], 0.0)
        return jnp.dot(h, aW3_ref[...], preferred_element_type=jnp.float32) \
            + ab3_ref[...][None, :]

    def weight_store(bb, xx, s):
        s = s - jnp.max(s)
        e = jnp.exp(s)
        w = e / jnp.sum(e)
        wf = xx * w
        wf_s[bb] = wf
        gc_s[bb:bb + 1, :] = jnp.sum(wf, axis=0, keepdims=True)
        ds_s[bb:bb + 1, :] = jnp.zeros((1, _C), jnp.float32)

    @pl.when((b == 0) & (i == 0))
    def _attention():
        for bb in range(_B):
            xx = x_ref[bb]
            weight_store(bb, xx, scores(xx))

    wf_b = wf_s[pl.ds(b, 1)][0]
    m = jnp.dot(a_ref[0], wf_b, preferred_element_type=jnp.float32)
    wfr = wf_s[pl.ds(b, 1), pl.ds(i * _TM, _TM), :][0]
    ds_s[pl.ds(b, 1), :] += jnp.sum(m * wfr, axis=0, keepdims=True)

    @pl.when((b == _B - 1) & (i == _NT - 1))
    def _global_mlp():
        gc = gc_s[...]
        cs = jnp.sum(ds_s[...], axis=1, keepdims=True) * (1.0 / _N)
        hin = jnp.concatenate([gc, gc + 0.001 * cs], axis=0)
        h = jnp.maximum(
            jnp.dot(hin, gW1_ref[...], preferred_element_type=jnp.float32)
            + gb1_ref[...][None, :], 0.0)
        h = jnp.maximum(
            jnp.dot(h, gW2_ref[...], preferred_element_type=jnp.float32)
            + gb2_ref[...][None, :], 0.0)
        g = jnp.maximum(
            jnp.dot(h, gW3_ref[...], preferred_element_type=jnp.float32)
            + gb3_ref[...][None, :], 0.0)
        out_ref[...] = g[:_B] + 0.002 * g[_B:]


def kernel(x, adjacency, aW1, ab1, aW2, ab2, aW3, ab3,
           gW1, gb1, gW2, gb2, gW3, gb3):
    const = lambda shape: pl.BlockSpec(shape, lambda b, i: tuple(0 for _ in shape))
    out = pl.pallas_call(
        _fused_kernel,
        grid=(_B, _NT),
        in_specs=[
            const((_B, _N, _C)),
            pl.BlockSpec((1, _TM, _N), lambda b, i: (b, i, 0)),
            const((_C, 128)), const((128,)),
            const((128, 64)), const((64,)),
            const((64, 1)), const((1,)),
            const((_C, 1024)), const((1024,)),
            const((1024, 1024)), const((1024,)),
            const((1024, _C)), const((_C,)),
        ],
        out_specs=pl.BlockSpec((_B, _C), lambda b, i: (0, 0)),
        out_shape=jax.ShapeDtypeStruct((_B, _C), jnp.float32),
        scratch_shapes=[
            pltpu.VMEM((_B, _N, _C), jnp.float32),
            pltpu.VMEM((_B, _C), jnp.float32),
            pltpu.VMEM((_B, _C), jnp.float32),
        ],
    )(x, adjacency, aW1, ab1, aW2, ab2, aW3, ab3,
      gW1, gb1, gW2, gb2, gW3, gb3)
    return out


# speedup vs baseline: 1.0499x; 1.0117x over previous
import jax
import jax.numpy as jnp
from jax.experimental import pallas as pl
from jax.experimental.pallas import tpu as pltpu

_B, _N, _C = 2, 4096, 128
_TM = 512
_NT = _N // _TM
_T = _B * _NT
_NBUF = 3


def _fused_kernel(x_ref, a_ref, aW1_ref, ab1_ref, aW2_ref, ab2_ref,
                  aW3_ref, ab3_ref, gW1_ref, gb1_ref, gW2_ref, gb2_ref,
                  gW3_ref, gb3_ref, out_ref, wf_s, gc_s, ds_s, a_buf, sem):
    b = pl.program_id(0)
    i = pl.program_id(1)
    t = b * _NT + i

    def a_copy(tt, slot):
        bb = jax.lax.div(tt, _NT)
        ii = jax.lax.rem(tt, _NT)
        return pltpu.make_async_copy(
            a_ref.at[bb, pl.ds(ii * _TM, _TM), :], a_buf.at[slot], sem.at[slot])

    @pl.when(t == 0)
    def _prefetch():
        for k in range(_NBUF):
            a_copy(k, k).start()

    def scores(xx):
        h = jnp.maximum(
            jnp.dot(xx, aW1_ref[...], preferred_element_type=jnp.float32)
            + ab1_ref[...][None, :], 0.0)
        h = jnp.maximum(
            jnp.dot(h, aW2_ref[...], preferred_element_type=jnp.float32)
            + ab2_ref[...][None, :], 0.0)
        return jnp.dot(h, aW3_ref[...], preferred_element_type=jnp.float32) \
            + ab3_ref[...][None, :]

    def weight_store(bb, xx, s):
        s = s - jnp.max(s)
        e = jnp.exp(s)
        w = e / jnp.sum(e)
        wf = xx * w
        wf_s[bb] = wf
        gc_s[bb:bb + 1, :] = jnp.sum(wf, axis=0, keepdims=True)
        ds_s[bb:bb + 1, :] = jnp.zeros((1, _C), jnp.float32)

    @pl.when(t == 0)
    def _attention():
        for bb in range(_B):
            xx = x_ref[bb]
            weight_store(bb, xx, scores(xx))

    slot = jax.lax.rem(t, _NBUF)
    a_copy(t, slot).wait()
    wf_b = wf_s[pl.ds(b, 1)][0]
    m = jnp.dot(a_buf[pl.ds(slot, 1)][0], wf_b,
                preferred_element_type=jnp.float32)
    wfr = wf_s[pl.ds(b, 1), pl.ds(i * _TM, _TM), :][0]
    ds_s[pl.ds(b, 1), :] += jnp.sum(m * wfr, axis=0, keepdims=True)

    @pl.when(t + _NBUF < _T)
    def _next_copy():
        a_copy(t + _NBUF, slot).start()

    @pl.when(t == _T - 1)
    def _global_mlp():
        gc = gc_s[...]
        cs = jnp.sum(ds_s[...], axis=1, keepdims=True) * (1.0 / _N)
        hin = jnp.concatenate([gc, gc + 0.001 * cs], axis=0)
        h = jnp.maximum(
            jnp.dot(hin, gW1_ref[...], preferred_element_type=jnp.float32)
            + gb1_ref[...][None, :], 0.0)
        h = jnp.maximum(
            jnp.dot(h, gW2_ref[...], preferred_element_type=jnp.float32)
            + gb2_ref[...][None, :], 0.0)
        g = jnp.maximum(
            jnp.dot(h, gW3_ref[...], preferred_element_type=jnp.float32)
            + gb3_ref[...][None, :], 0.0)
        out_ref[...] = g[:_B] + 0.002 * g[_B:]


def kernel(x, adjacency, aW1, ab1, aW2, ab2, aW3, ab3,
           gW1, gb1, gW2, gb2, gW3, gb3):
    const = lambda shape: pl.BlockSpec(shape, lambda b, i: tuple(0 for _ in shape))
    out = pl.pallas_call(
        _fused_kernel,
        grid=(_B, _NT),
        in_specs=[
            const((_B, _N, _C)),
            pl.BlockSpec(memory_space=pltpu.MemorySpace.HBM),
            const((_C, 128)), const((128,)),
            const((128, 64)), const((64,)),
            const((64, 1)), const((1,)),
            const((_C, 1024)), const((1024,)),
            const((1024, 1024)), const((1024,)),
            const((1024, _C)), const((_C,)),
        ],
        out_specs=pl.BlockSpec((_B, _C), lambda b, i: (0, 0)),
        out_shape=jax.ShapeDtypeStruct((_B, _C), jnp.float32),
        scratch_shapes=[
            pltpu.VMEM((_B, _N, _C), jnp.float32),
            pltpu.VMEM((_B, _C), jnp.float32),
            pltpu.VMEM((_B, _C), jnp.float32),
            pltpu.VMEM((_NBUF, _TM, _N), jnp.float32),
            pltpu.SemaphoreType.DMA((_NBUF,)),
        ],
    )(x, adjacency, aW1, ab1, aW2, ab2, aW3, ab3,
      gW1, gb1, gW2, gb2, gW3, gb3)
    return out
